# Initial kernel scaffold; baseline (speedup 1.0000x reference)
import jax
import jax.numpy as jnp
from jax.experimental import pallas as pl


def kernel(inputs, targets):
    raise NotImplementedError("write your pallas kernel here")


# speedup vs baseline: 16.4056x; 16.4056x over previous
import numpy as np
import jax
import jax.numpy as jnp
from jax import lax
from jax.experimental import pallas as pl
from jax.experimental.pallas import tpu as pltpu
from jax.experimental.pallas import tpu_sc as plsc

_RGB_FROM_HED = np.array(
    [[0.65, 0.7, 0.29], [0.07, 0.99, 0.11], [0.27, 0.57, 0.78]], dtype=np.float32
)
_HED_FROM_RGB = np.linalg.inv(_RGB_FROM_HED)
_COEFFS = np.array([0.2125, 0.7154, 0.0721], dtype=np.float32)
_ALPHA = 1.8
_ADJUST_CAL = np.float32(10.0 ** (-(np.e ** (1.0 / _ALPHA))))
_THRESH_FOD = 0.15
_THRESH_MASK = 0.68
_LOG_ADJUST = np.float32(np.log(1e-6))
_NUM_BINS = 20
_BUCKET_W = np.float32(np.e / _NUM_BINS)

_B, _C, _H, _W = 16, 3, 512, 512
_SROWS = 8


def _b16(c):
    import ml_dtypes
    return float(np.array(c, dtype=np.float32).astype(ml_dtypes.bfloat16)
                 .astype(np.float32))


def _rtne_bf16(v):
    u = jax.lax.bitcast_convert_type(v, jnp.uint32)
    u = u + np.uint32(0x7FFF) + ((u >> 16) & np.uint32(1))
    return jax.lax.bitcast_convert_type(u & np.uint32(0xFFFF0000), jnp.float32)


_W0 = _b16(_HED_FROM_RGB[0, 2])
_W1 = _b16(_HED_FROM_RGB[1, 2])
_W2 = _b16(_HED_FROM_RGB[2, 2])
_NEG_LOG_ADJUST = float(-_LOG_ADJUST)
_K0 = _b16(_RGB_FROM_HED[2, 0])
_K1 = _b16(_RGB_FROM_HED[2, 1])
_K2 = _b16(_RGB_FROM_HED[2, 2])
_C0 = _b16(_COEFFS[0])
_C1 = _b16(_COEFFS[1])
_C2 = _b16(_COEFFS[2])


def _process(rgb):
    r = jnp.maximum(rgb[0], 1e-6)
    g = jnp.maximum(rgb[1], 1e-6)
    b = jnp.maximum(rgb[2], 1e-6)
    lr = _rtne_bf16(jnp.log(r) / _LOG_ADJUST)
    lg = _rtne_bf16(jnp.log(g) / _LOG_ADJUST)
    lb = _rtne_bf16(jnp.log(b) / _LOG_ADJUST)
    s = jnp.maximum((lr * _W0 + lg * _W1) + lb * _W2, 0.0)
    t = _rtne_bf16(s * _NEG_LOG_ADJUST)
    c0 = jnp.exp(-(t * _K0))
    c1 = jnp.exp(-(t * _K1))
    c2 = jnp.exp(-(t * _K2))
    grey = (_rtne_bf16(c0) * _C0 + _rtne_bf16(c1) * _C1) + _rtne_bf16(c2) * _C2
    fod = jnp.log10(1.0 / (grey + _ADJUST_CAL))
    fod = jnp.where(fod < 0.0, 0.0, fod)
    fodp = jnp.where(
        fod > 0.0, jnp.exp(_ALPHA * jnp.log(jnp.maximum(fod, 1e-37))), 0.0
    )
    mask = jnp.where(fodp < _THRESH_MASK, 0.0, 1.0)
    relu = jnp.where(fodp < _THRESH_FOD, 0.0, fodp)
    relu_cols = jnp.sum(relu.reshape(4, _H // 4, _W), axis=1)
    return mask, fodp, relu_cols


def _tc_kernel(x_ref, y_ref, imask_ref, tmask_ref, ifod_ref, tfod_ref,
               istats_ref, tstats_ref):
    b = pl.program_id(0)
    im, ifo, ir = _process(x_ref[0])
    tm, tfo, tr = _process(y_ref[0])
    imask_ref[0] = im
    tmask_ref[0] = tm
    ifod_ref[0] = ifo
    tfod_ref[0] = tfo
    istats_ref[pl.ds(b, 1), 0:4, :] = ir[None]
    tstats_ref[pl.ds(b, 1), 0:4, :] = tr[None]


def _tc_call(inputs, targets):
    out_shapes = (
        jax.ShapeDtypeStruct((_B, _H, _W), jnp.float32),
        jax.ShapeDtypeStruct((_B, _H, _W), jnp.float32),
        jax.ShapeDtypeStruct((_B, _H, _W), jnp.float32),
        jax.ShapeDtypeStruct((_B, _H, _W), jnp.float32),
        jax.ShapeDtypeStruct((_B, _SROWS, _W), jnp.float32),
        jax.ShapeDtypeStruct((_B, _SROWS, _W), jnp.float32),
    )
    in_spec = pl.BlockSpec((1, _C, _H, _W), lambda b: (b, 0, 0, 0))
    mask_spec = pl.BlockSpec((1, _H, _W), lambda b: (b, 0, 0))
    return pl.pallas_call(
        _tc_kernel,
        grid=(_B,),
        in_specs=[in_spec, in_spec],
        out_specs=(
            mask_spec, mask_spec, mask_spec, mask_spec,
            pl.BlockSpec((_B, _SROWS, _W), lambda b: (0, 0, 0)),
            pl.BlockSpec((_B, _SROWS, _W), lambda b: (0, 0, 0)),
        ),
        out_shape=out_shapes,
        compiler_params=pltpu.CompilerParams(
            dimension_semantics=("arbitrary",),
        ),
    )(inputs, targets)


_NW = 32
_PER = (_H * _W) // _NW
_HLANES = 64
_INV_W = float(np.float32(1.0) / _BUCKET_W)


def _sc_hist_call(ifod_flat, tfod_flat):
    mesh = plsc.VectorSubcoreMesh(core_axis_name="c", subcore_axis_name="s")

    def body(ifod_hbm, tfod_hbm, out_hbm, buf, hist):
        wid = lax.axis_index("s") * 2 + lax.axis_index("c")
        zero = jnp.zeros((16,), jnp.float32)
        for c in range((_B * _HLANES) // 16):
            hist[pl.ds(c * 16, 16)] = zero
        for tsel in range(2):
            src = ifod_hbm if tsel == 0 else tfod_hbm
            for b in range(_B):
                pltpu.sync_copy(
                    src.at[pl.ds(b * (_H * _W) + wid * _PER, _PER)], buf)
                base = b * _HLANES + tsel * 32

                def step(i, carry, base=base):
                    v = buf[pl.ds(pl.multiple_of(i * 16, 16), 16)]
                    iv = jnp.minimum(v * _INV_W, float(_NUM_BINS - 1))
                    idx = iv.astype(jnp.int32) + base
                    plsc.addupdate_scatter(hist, [idx], v)
                    return carry

                lax.fori_loop(0, _PER // 16, step, 0)
        pltpu.sync_copy(hist, out_hbm.at[wid])

    k = pl.kernel(
        body,
        mesh=mesh,
        compiler_params=pltpu.CompilerParams(needs_layout_passes=False),
        out_type=jax.ShapeDtypeStruct((_NW, _B * _HLANES), jnp.float32),
        scratch_types=[
            pltpu.VMEM((_PER,), jnp.float32),
            pltpu.VMEM((_B * _HLANES,), jnp.float32),
        ],
    )
    return k(ifod_flat, tfod_flat)


def _combine_kernel(istats_ref, tstats_ref, scat_ref, loss_ref):
    hw = float(_H * _W)
    hs = jnp.sum(scat_ref[...], axis=0)
    hi = hs[:, 0:_NUM_BINS]
    ht = hs[:, 32:32 + _NUM_BINS]
    dh = (hi - ht) * (1.0 / hw)
    mlpa_histo = jnp.sum(dh * dh, axis=1, keepdims=True) / float(_B)
    ist = istats_ref[...]
    tst = tstats_ref[...]
    ai = jnp.sum(jnp.sum(ist[:, 0:4, :], axis=-1), axis=-1, keepdims=True)
    at = jnp.sum(jnp.sum(tst[:, 0:4, :], axis=-1), axis=-1, keepdims=True)
    diff = ai - at
    da = diff * (1.0 / hw)
    mlpa_avg = da * da
    cond = jnp.logical_and(diff >= at * -0.4, diff <= at * 0.4)
    loss = jnp.sum(jnp.where(cond, mlpa_histo, mlpa_avg + mlpa_histo))
    dbc = (ist[:, 0:4, :] - tst[:, 0:4, :]) * (16.0 / hw)
    blk = 0.0
    for j in range(4):
        dj = jnp.sum(dbc[:, :, 128 * j:128 * (j + 1)], axis=-1)
        blk = blk + jnp.sum(dj * dj)
    loss = loss + blk / float(_B * 16)
    loss_ref[...] = jnp.full((1, 1), loss, jnp.float32)


def _combine_call(istats, tstats, scat):
    return pl.pallas_call(
        _combine_kernel,
        out_shape=jax.ShapeDtypeStruct((1, 1), jnp.float32),
    )(istats, tstats, scat)


def kernel(inputs, targets):
    imask, tmask, ifod, tfod, ist, tst = _tc_call(inputs, targets)
    scat = _sc_hist_call(ifod.reshape(-1), tfod.reshape(-1))
    loss = _combine_call(ist, tst, scat.reshape(_NW, _B, _HLANES))
    return (loss[0, 0], imask, tmask)


# speedup vs baseline: 21.5809x; 1.3155x over previous
import numpy as np
import jax
import jax.numpy as jnp
from jax import lax
from jax.experimental import pallas as pl
from jax.experimental.pallas import tpu as pltpu

_RGB_FROM_HED = np.array(
    [[0.65, 0.7, 0.29], [0.07, 0.99, 0.11], [0.27, 0.57, 0.78]], dtype=np.float32
)
_HED_FROM_RGB = np.linalg.inv(_RGB_FROM_HED)
_COEFFS = np.array([0.2125, 0.7154, 0.0721], dtype=np.float32)
_ALPHA = 1.8
_ADJUST_CAL = np.float32(10.0 ** (-(np.e ** (1.0 / _ALPHA))))
_THRESH_FOD = 0.15
_THRESH_MASK = 0.68
_LOG_ADJUST = np.float32(np.log(1e-6))
_NUM_BINS = 20
_BUCKET_W = np.float32(np.e / _NUM_BINS)

_B, _C, _H, _W = 16, 3, 512, 512
_HC = 512
_SROWS = 24

def _b16(c):
    import ml_dtypes
    return float(np.array(c, dtype=np.float32).astype(ml_dtypes.bfloat16)
                 .astype(np.float32))


def _rtne_bf16(v):
    u = jax.lax.bitcast_convert_type(v, jnp.uint32)
    u = u + np.uint32(0x7FFF) + ((u >> 16) & np.uint32(1))
    return jax.lax.bitcast_convert_type(u & np.uint32(0xFFFF0000), jnp.float32)


_W0 = _b16(_HED_FROM_RGB[0, 2])
_W1 = _b16(_HED_FROM_RGB[1, 2])
_W2 = _b16(_HED_FROM_RGB[2, 2])
_NEG_LOG_ADJUST = float(-_LOG_ADJUST)
_K0 = _b16(_RGB_FROM_HED[2, 0])
_K1 = _b16(_RGB_FROM_HED[2, 1])
_K2 = _b16(_RGB_FROM_HED[2, 2])
_C0 = _b16(_COEFFS[0])
_C1 = _b16(_COEFFS[1])
_C2 = _b16(_COEFFS[2])


def _process(rgb):
    r = jnp.maximum(rgb[0], 1e-6)
    g = jnp.maximum(rgb[1], 1e-6)
    b = jnp.maximum(rgb[2], 1e-6)
    lr = _rtne_bf16(jnp.log(r) / _LOG_ADJUST)
    lg = _rtne_bf16(jnp.log(g) / _LOG_ADJUST)
    lb = _rtne_bf16(jnp.log(b) / _LOG_ADJUST)
    s = jnp.maximum((lr * _W0 + lg * _W1) + lb * _W2, 0.0)
    t = _rtne_bf16(s * _NEG_LOG_ADJUST)
    c0 = jnp.exp(-(t * _K0))
    c1 = jnp.exp(-(t * _K1))
    c2 = jnp.exp(-(t * _K2))
    grey = (_rtne_bf16(c0) * _C0 + _rtne_bf16(c1) * _C1) + _rtne_bf16(c2) * _C2
    fod = jnp.log10(1.0 / (grey + _ADJUST_CAL))
    fod = jnp.where(fod < 0.0, 0.0, fod)
    fodp = jnp.where(
        fod > 0.0, jnp.exp(_ALPHA * jnp.log(jnp.maximum(fod, 1e-37))), 0.0
    )
    mask = jnp.where(fodp < _THRESH_MASK, 0.0, 1.0)
    relu = jnp.where(fodp < _THRESH_FOD, 0.0, fodp)
    def _colsum(x):
        return lax.dot_general(
            jnp.ones((1, x.shape[0]), jnp.float32), x,
            (((1,), (0,)), ((), ())),
            precision=lax.Precision.HIGHEST,
            preferred_element_type=jnp.float32,
        )

    hist = jnp.concatenate(
        [_colsum(fodp)]
        + [
            _colsum(jnp.where(fodp >= float(j) * _BUCKET_W, fodp, 0.0))
            for j in range(1, _NUM_BINS)
        ]
        + [_colsum(relu.reshape(4, _H // 4, _W)[k]) for k in range(4)],
        axis=0,
    )
    return mask, hist, None


def _mlpa_kernel(x_ref, y_ref, imask_ref, tmask_ref, istats_ref, tstats_ref,
                 loss_ref):
    b = pl.program_id(0)

    im, ih, _ = _process(x_ref[0])
    tm, th, _ = _process(y_ref[0])
    imask_ref[0] = im
    tmask_ref[0] = tm
    istats_ref[pl.ds(b, 1), :, :] = ih[None]
    tstats_ref[pl.ds(b, 1), :, :] = th[None]

    @pl.when(b == _B - 1)
    def _finish():
        hw = float(_H * _W)
        ist = istats_ref[...]
        tst = tstats_ref[...]
        si = jnp.sum(ist[:, 0:_NUM_BINS, :], axis=-1)
        st = jnp.sum(tst[:, 0:_NUM_BINS, :], axis=-1)
        hi = jnp.concatenate([si[:, :-1] - si[:, 1:], si[:, -1:]], axis=1)
        ht = jnp.concatenate([st[:, :-1] - st[:, 1:], st[:, -1:]], axis=1)
        dh = (hi - ht) * (1.0 / hw)
        mlpa_histo = jnp.sum(dh * dh, axis=1, keepdims=True) / float(_B)
        ai = jnp.sum(jnp.sum(ist[:, _NUM_BINS:_SROWS, :], axis=-1), axis=-1,
                     keepdims=True)
        at = jnp.sum(jnp.sum(tst[:, _NUM_BINS:_SROWS, :], axis=-1), axis=-1,
                     keepdims=True)
        diff = ai - at
        da = diff * (1.0 / hw)
        mlpa_avg = da * da
        cond = jnp.logical_and(diff >= at * -0.4, diff <= at * 0.4)
        loss = jnp.sum(jnp.where(cond, mlpa_histo, mlpa_avg + mlpa_histo))
        dbc = (ist[:, _NUM_BINS:_SROWS, :] - tst[:, _NUM_BINS:_SROWS, :]) * (
            16.0 / hw
        )
        blk = 0.0
        for j in range(4):
            dj = jnp.sum(dbc[:, :, 128 * j:128 * (j + 1)], axis=-1)
            blk = blk + jnp.sum(dj * dj)
        loss = loss + blk / float(_B * 16)
        loss_ref[...] = jnp.full((1, 1), loss, jnp.float32)


def _run(inputs, targets, interpret=False):
    out_shapes = (
        jax.ShapeDtypeStruct((_B, _H, _W), jnp.float32),
        jax.ShapeDtypeStruct((_B, _H, _W), jnp.float32),
        jax.ShapeDtypeStruct((_B, _SROWS, _W), jnp.float32),
        jax.ShapeDtypeStruct((_B, _SROWS, _W), jnp.float32),
        jax.ShapeDtypeStruct((1, 1), jnp.float32),
    )
    in_spec = pl.BlockSpec((1, _C, _H, _W), lambda b: (b, 0, 0, 0))
    grid = (_B,)
    return pl.pallas_call(
        _mlpa_kernel,
        grid=grid,
        in_specs=[in_spec, in_spec],
        out_specs=(
            pl.BlockSpec((1, _H, _W), lambda b: (b, 0, 0)),
            pl.BlockSpec((1, _H, _W), lambda b: (b, 0, 0)),
            pl.BlockSpec((_B, _SROWS, _W), lambda b: (0, 0, 0)),
            pl.BlockSpec((_B, _SROWS, _W), lambda b: (0, 0, 0)),
            pl.BlockSpec((1, 1), lambda b: (0, 0)),
        ),
        out_shape=out_shapes,
        compiler_params=pltpu.CompilerParams(
            dimension_semantics=("arbitrary",),
        ),
        interpret=interpret,
    )(inputs, targets)


def kernel(inputs, targets):
    imask, tmask, _, _, loss = _run(inputs, targets)
    return (loss[0, 0], imask, tmask)


# speedup vs baseline: 51.3682x; 2.3803x over previous
import numpy as np
import jax
import jax.numpy as jnp
from jax import lax
from jax.experimental import pallas as pl
from jax.experimental.pallas import tpu as pltpu

_RGB_FROM_HED = np.array(
    [[0.65, 0.7, 0.29], [0.07, 0.99, 0.11], [0.27, 0.57, 0.78]], dtype=np.float32
)
_HED_FROM_RGB = np.linalg.inv(_RGB_FROM_HED)
_COEFFS = np.array([0.2125, 0.7154, 0.0721], dtype=np.float32)
_ALPHA = 1.8
_ADJUST_CAL = np.float32(10.0 ** (-(np.e ** (1.0 / _ALPHA))))
_THRESH_FOD = 0.15
_THRESH_MASK = 0.68
_LOG_ADJUST = np.float32(np.log(1e-6))
_NUM_BINS = 20
_BUCKET_W = np.float32(np.e / _NUM_BINS)

_B, _C, _H, _W = 16, 3, 512, 512
_HC = 512
_SROWS = 24

def _b16(c):
    import ml_dtypes
    return float(np.array(c, dtype=np.float32).astype(ml_dtypes.bfloat16)
                 .astype(np.float32))


def _rtne_bf16(v):
    u = jax.lax.bitcast_convert_type(v, jnp.uint32)
    u = u + np.uint32(0x7FFF) + ((u >> 16) & np.uint32(1))
    return jax.lax.bitcast_convert_type(u & np.uint32(0xFFFF0000), jnp.float32)


_W0 = _b16(_HED_FROM_RGB[0, 2])
_W1 = _b16(_HED_FROM_RGB[1, 2])
_W2 = _b16(_HED_FROM_RGB[2, 2])
_NEG_LOG_ADJUST = float(-_LOG_ADJUST)
_K0 = _b16(_RGB_FROM_HED[2, 0])
_K1 = _b16(_RGB_FROM_HED[2, 1])
_K2 = _b16(_RGB_FROM_HED[2, 2])
_C0 = _b16(_COEFFS[0])
_C1 = _b16(_COEFFS[1])
_C2 = _b16(_COEFFS[2])


def _process(rgb):
    r = jnp.maximum(rgb[0], 1e-6)
    g = jnp.maximum(rgb[1], 1e-6)
    b = jnp.maximum(rgb[2], 1e-6)
    lr = _rtne_bf16(jnp.log(r) / _LOG_ADJUST)
    lg = _rtne_bf16(jnp.log(g) / _LOG_ADJUST)
    lb = _rtne_bf16(jnp.log(b) / _LOG_ADJUST)
    s = jnp.maximum((lr * _W0 + lg * _W1) + lb * _W2, 0.0)
    t = _rtne_bf16(s * _NEG_LOG_ADJUST)
    c0 = jnp.exp(-(t * _K0))
    c1 = jnp.exp(-(t * _K1))
    c2 = jnp.exp(-(t * _K2))
    grey = (_rtne_bf16(c0) * _C0 + _rtne_bf16(c1) * _C1) + _rtne_bf16(c2) * _C2
    fod = jnp.log10(1.0 / (grey + _ADJUST_CAL))
    fod = jnp.where(fod < 0.0, 0.0, fod)
    fodp = jnp.where(
        fod > 0.0, jnp.exp(_ALPHA * jnp.log(jnp.maximum(fod, 1e-37))), 0.0
    )
    mask = jnp.where(fodp < _THRESH_MASK, 0.0, 1.0)
    relu = jnp.where(fodp < _THRESH_FOD, 0.0, fodp)
    hist = jnp.concatenate(
        [jnp.sum(fodp, axis=0, keepdims=True)]
        + [
            jnp.sum(jnp.where(fodp >= float(j) * _BUCKET_W, fodp, 0.0),
                    axis=0, keepdims=True)
            for j in range(1, _NUM_BINS)
        ],
        axis=0,
    )
    relu_cols = jnp.sum(relu.reshape(4, _H // 4, _W), axis=1)
    return mask, jnp.concatenate([hist, relu_cols], axis=0), None


def _mlpa_kernel(x_ref, y_ref, imask_ref, tmask_ref, istats_ref, tstats_ref,
                 loss_ref):
    b = pl.program_id(0)

    im, ih, _ = _process(x_ref[0])
    tm, th, _ = _process(y_ref[0])
    imask_ref[0] = im
    tmask_ref[0] = tm
    istats_ref[pl.ds(b, 1), :, :] = ih[None]
    tstats_ref[pl.ds(b, 1), :, :] = th[None]

    @pl.when(b == _B - 1)
    def _finish():
        hw = float(_H * _W)
        ist = istats_ref[...]
        tst = tstats_ref[...]
        si = jnp.sum(ist[:, 0:_NUM_BINS, :], axis=-1)
        st = jnp.sum(tst[:, 0:_NUM_BINS, :], axis=-1)
        hi = jnp.concatenate([si[:, :-1] - si[:, 1:], si[:, -1:]], axis=1)
        ht = jnp.concatenate([st[:, :-1] - st[:, 1:], st[:, -1:]], axis=1)
        dh = (hi - ht) * (1.0 / hw)
        mlpa_histo = jnp.sum(dh * dh, axis=1, keepdims=True) / float(_B)
        ai = jnp.sum(jnp.sum(ist[:, _NUM_BINS:_SROWS, :], axis=-1), axis=-1,
                     keepdims=True)
        at = jnp.sum(jnp.sum(tst[:, _NUM_BINS:_SROWS, :], axis=-1), axis=-1,
                     keepdims=True)
        diff = ai - at
        da = diff * (1.0 / hw)
        mlpa_avg = da * da
        cond = jnp.logical_and(diff >= at * -0.4, diff <= at * 0.4)
        loss = jnp.sum(jnp.where(cond, mlpa_histo, mlpa_avg + mlpa_histo))
        dbc = (ist[:, _NUM_BINS:_SROWS, :] - tst[:, _NUM_BINS:_SROWS, :]) * (
            16.0 / hw
        )
        blk = 0.0
        for j in range(4):
            dj = jnp.sum(dbc[:, :, 128 * j:128 * (j + 1)], axis=-1)
            blk = blk + jnp.sum(dj * dj)
        loss = loss + blk / float(_B * 16)
        loss_ref[...] = jnp.full((1, 1), loss, jnp.float32)


def _run(inputs, targets, interpret=False):
    out_shapes = (
        jax.ShapeDtypeStruct((_B, _H, _W), jnp.float32),
        jax.ShapeDtypeStruct((_B, _H, _W), jnp.float32),
        jax.ShapeDtypeStruct((_B, _SROWS, _W), jnp.float32),
        jax.ShapeDtypeStruct((_B, _SROWS, _W), jnp.float32),
        jax.ShapeDtypeStruct((1, 1), jnp.float32),
    )
    in_spec = pl.BlockSpec((1, _C, _H, _W), lambda b: (b, 0, 0, 0))
    grid = (_B,)
    return pl.pallas_call(
        _mlpa_kernel,
        grid=grid,
        in_specs=[in_spec, in_spec],
        out_specs=(
            pl.BlockSpec((1, _H, _W), lambda b: (b, 0, 0)),
            pl.BlockSpec((1, _H, _W), lambda b: (b, 0, 0)),
            pl.BlockSpec((_B, _SROWS, _W), lambda b: (0, 0, 0)),
            pl.BlockSpec((_B, _SROWS, _W), lambda b: (0, 0, 0)),
            pl.BlockSpec((1, 1), lambda b: (0, 0)),
        ),
        out_shape=out_shapes,
        compiler_params=pltpu.CompilerParams(
            dimension_semantics=("arbitrary",),
        ),
        interpret=interpret,
    )(inputs, targets)


def kernel(inputs, targets):
    imask, tmask, _, _, loss = _run(inputs, targets)
    return (loss[0, 0], imask, tmask)
